# Initial kernel scaffold; baseline (speedup 1.0000x reference)
import jax
import jax.numpy as jnp
from jax.experimental import pallas as pl


def kernel(input, g, h0, lamda, alpha, l, W):
    raise NotImplementedError("write your pallas kernel here")


# speedup vs baseline: 4.6010x; 4.6010x over previous
import functools

import jax
import jax.numpy as jnp
from jax import lax
from jax.experimental import pallas as pl
from jax.experimental.pallas import tpu as pltpu
from jax.experimental.pallas import tpu_sc as plsc

N_NODES = 10000
D = 128
N_EDGES = 320000

NC = 2
NS = 16
NW = NC * NS

CHUNK = 128
EPT = ((N_EDGES // NW + CHUNK - 1) // CHUNK) * CHUNK
NCH = EPT // CHUNK
PE = EPT * NW

DEG_N = 10240
DEG_SL = DEG_N // NS
NP = 10112
AGG_SL = NP // NS

_mesh = plsc.VectorSubcoreMesh(
    core_axis_name="c", subcore_axis_name="s", num_cores=NC, num_subcores=NS
)


def _deg_body(srcp, dstp, consts, out, sidx, didx, ones_v, deg_s, deg_d):
    c = lax.axis_index("c")
    s = lax.axis_index("s")
    off = s * DEG_SL
    pltpu.sync_copy(consts.at[pl.ds(off, DEG_SL)], deg_s.at[pl.ds(off, DEG_SL)])
    pltpu.sync_copy(consts.at[pl.ds(off, DEG_SL)], deg_d.at[pl.ds(off, DEG_SL)])
    pltpu.sync_copy(consts.at[pl.ds(DEG_N, CHUNK)], ones_v)
    m = c * NS + s
    pltpu.sync_copy(srcp.at[m], sidx)
    pltpu.sync_copy(dstp.at[m], didx)
    plsc.subcore_barrier()

    def body(j, carry):
        pltpu.sync_copy(ones_v, deg_s.at[sidx.at[j]], add=True)
        pltpu.sync_copy(ones_v, deg_d.at[didx.at[j]], add=True)
        return carry

    lax.fori_loop(0, NCH, body, 0)
    plsc.subcore_barrier()
    pltpu.sync_copy(deg_s.at[pl.ds(off, DEG_SL)], out.at[c, 0, pl.ds(off, DEG_SL)])
    pltpu.sync_copy(deg_d.at[pl.ds(off, DEG_SL)], out.at[c, 1, pl.ds(off, DEG_SL)])


_deg_kernel = pl.kernel(
    _deg_body,
    out_type=jax.ShapeDtypeStruct((NC, 2, DEG_N), jnp.float32),
    mesh=_mesh,
    scratch_types=[
        pltpu.VMEM((NCH, CHUNK), jnp.int32),
        pltpu.VMEM((NCH, CHUNK), jnp.int32),
        pltpu.VMEM((CHUNK,), jnp.float32),
        pltpu.VMEM_SHARED((DEG_N,), jnp.float32),
        pltpu.VMEM_SHARED((DEG_N,), jnp.float32),
    ],
)


def _agg_body(xn, srcp, dstp, zrows, out, sidx, didx, rows, agg):
    c = lax.axis_index("c")
    s = lax.axis_index("s")
    off = s * AGG_SL
    pltpu.sync_copy(zrows, agg.at[pl.ds(off, AGG_SL)])
    m = c * NS + s
    pltpu.sync_copy(srcp.at[m], sidx)
    pltpu.sync_copy(dstp.at[m], didx)
    plsc.subcore_barrier()

    def body(j, carry):
        pltpu.sync_copy(xn.at[sidx.at[j]], rows)
        pltpu.sync_copy(rows, agg.at[didx.at[j]], add=True)
        return carry

    lax.fori_loop(0, NCH, body, 0)
    plsc.subcore_barrier()
    pltpu.sync_copy(agg.at[pl.ds(off, AGG_SL)], out.at[c, pl.ds(off, AGG_SL)])


_agg_kernel = pl.kernel(
    _agg_body,
    out_type=jax.ShapeDtypeStruct((NC, NP, D), jnp.float32),
    mesh=_mesh,
    scratch_types=[
        pltpu.VMEM((NCH, CHUNK), jnp.int32),
        pltpu.VMEM((NCH, CHUNK), jnp.int32),
        pltpu.VMEM((CHUNK, D), jnp.float32),
        pltpu.VMEM_SHARED((NP, D), jnp.float32),
    ],
)


def _scale_body(x_ref, degt_ref, xn_ref):
    d = degt_ref[...]
    ns = lax.rsqrt(jnp.maximum(d[:, 0:1] + d[:, 2:3], 1.0))
    xn_ref[:N_NODES, :] = x_ref[...] * ns[:N_NODES]
    xn_ref[N_NODES:, :] = jnp.zeros((NP - N_NODES, D), jnp.float32)


def _final_body(aggp_ref, degt_ref, h0_ref, w_ref, s_ref, out_ref):
    d = degt_ref[...]
    nd = lax.rsqrt(jnp.maximum(d[:, 1:2] + d[:, 3:4], 1.0))[:N_NODES]
    agg = aggp_ref[0, :N_NODES, :] + aggp_ref[1, :N_NODES, :]
    alpha = s_ref[0:1, 0:1]
    theta = s_ref[1:2, 0:1]
    sup = (1.0 - alpha) * (agg * nd) + alpha * h0_ref[...]
    mm = jnp.dot(sup, w_ref[...], preferred_element_type=jnp.float32)
    out_ref[...] = theta * mm + (1.0 - theta) * sup


def kernel(input, g, h0, lamda, alpha, l, W):
    x = input.astype(jnp.float32)
    src = g[0].astype(jnp.int32)
    dst = g[1].astype(jnp.int32)
    pad = jnp.full((PE - N_EDGES,), N_NODES, jnp.int32)
    srcp = jnp.concatenate([src, pad]).reshape(NW, NCH, CHUNK)
    dstp = jnp.concatenate([dst, pad]).reshape(NW, NCH, CHUNK)

    consts = jnp.concatenate(
        [jnp.zeros((DEG_N,), jnp.float32), jnp.ones((CHUNK,), jnp.float32)]
    )
    degp = _deg_kernel(srcp, dstp, consts)
    degt = degp.reshape(2 * NC, DEG_N).T

    xn = pl.pallas_call(
        _scale_body,
        out_shape=jax.ShapeDtypeStruct((NP, D), jnp.float32),
    )(x, degt)

    zrows = jnp.zeros((AGG_SL, D), jnp.float32)
    aggp = _agg_kernel(xn, srcp, dstp, zrows)

    theta = jnp.log(lamda / l + 1.0)
    scal = jnp.stack(
        [alpha.reshape(()).astype(jnp.float32), jnp.asarray(theta, jnp.float32)]
    ).reshape(2, 1)

    out = pl.pallas_call(
        _final_body,
        out_shape=jax.ShapeDtypeStruct((N_NODES, D), jnp.float32),
    )(aggp, degt, h0.astype(jnp.float32), W.astype(jnp.float32), scal)
    return out
